# Initial kernel scaffold; baseline (speedup 1.0000x reference)
import jax
import jax.numpy as jnp
from jax.experimental import pallas as pl


def kernel(X, word_table, pos_table):
    raise NotImplementedError("write your pallas kernel here")


# speedup vs baseline: 1.9702x; 1.9702x over previous
import functools

import jax
import jax.numpy as jnp
from jax import lax
from jax.experimental import pallas as pl
from jax.experimental.pallas import tpu as pltpu
from jax.experimental.pallas import tpu_sc as plsc

H = 256
V = 29
L = 12
LANES = 16

NC, NS = 2, 16
NW = NC * NS
B_TOTAL = 16384 * L
B_PER_W = B_TOTAL // NW
CHUNK = 128
N_CHUNKS = B_PER_W // CHUNK


def _fuse_body(word_ref, pos_ref, out_ref):
    out_ref[...] = word_ref[...][:, None, :] + pos_ref[...][None, :, :]


def _build_fused(word_table, pos_table):
    fused = pl.pallas_call(
        _fuse_body,
        out_shape=jax.ShapeDtypeStruct((V, L, H), jnp.float32),
    )(word_table, pos_table)
    return fused.reshape(V * L, H)


_sc_mesh = plsc.VectorSubcoreMesh(core_axis_name="c", subcore_axis_name="s")


@functools.partial(
    pl.kernel,
    out_type=jax.ShapeDtypeStruct((B_TOTAL, H), jnp.float32),
    mesh=_sc_mesh,
    scratch_types=[
        pltpu.VMEM((B_PER_W,), jnp.int32),
        pltpu.VMEM((B_PER_W,), jnp.int32),
        pltpu.VMEM((CHUNK, H), jnp.float32),
        pltpu.SemaphoreType.DMA,
    ],
)
def _sc_lookup(x_hbm, fused_hbm, out_hbm, x_v, idx_v, rows_v, gsem):
    wid = lax.axis_index("s") * NC + lax.axis_index("c")
    base = wid * B_PER_W

    pltpu.sync_copy(x_hbm.at[pl.ds(base, B_PER_W)], x_v)

    lane = lax.iota(jnp.int32, LANES)

    def idx_body(j, carry):
        off = j * LANES
        x = x_v[pl.ds(off, LANES)]
        idx_v[pl.ds(off, LANES)] = x * L + lax.rem(lane + off, L)
        return carry

    lax.fori_loop(0, B_PER_W // LANES, idx_body, 0)

    def chunk_body(g, carry):
        off = g * CHUNK
        pltpu.async_copy(
            fused_hbm.at[idx_v.at[pl.ds(off, CHUNK)]], rows_v, gsem
        ).wait()
        pltpu.sync_copy(rows_v, out_hbm.at[pl.ds(base + off, CHUNK)])
        return carry

    lax.fori_loop(0, N_CHUNKS, chunk_body, 0)


def kernel(X, word_table, pos_table):
    fused = _build_fused(word_table, pos_table)
    x_flat = X.reshape(-1).astype(jnp.int32)
    out = _sc_lookup(x_flat, fused)
    return out.reshape(X.shape[0], L, H)
